# Initial kernel scaffold; baseline (speedup 1.0000x reference)
import functools

import jax
import jax.numpy as jnp
from jax import lax
from jax.experimental import pallas as pl
from jax.experimental.pallas import tpu as pltpu
from jax.experimental.pallas import tpu_sc as plsc

NC = 2
NS = 16
NW = NC * NS

F = 16
CHUNK = 2560
KJ = CHUNK // 128


def _sc_mesh():
    return plsc.VectorSubcoreMesh(core_axis_name="c", subcore_axis_name="s")


def _make_deg_kernel(n_pad, e_pad):
    per_tile = e_pad // NW
    n_it = per_tile // CHUNK
    rows_per_tile = n_pad // NS

    @functools.partial(
        pl.kernel,
        out_type=jax.ShapeDtypeStruct((NC, n_pad), jnp.float32),
        mesh=_sc_mesh(),
        scratch_types=[
            pltpu.VMEM_SHARED((n_pad,), jnp.float32),
            pltpu.VMEM((KJ, 128), jnp.int32),
            pltpu.VMEM((CHUNK,), jnp.float32),
            pltpu.SemaphoreType.DMA,
        ],
    )
    def deg_kernel(dst_hbm, ones_hbm, zeros_hbm, out_hbm, acc, didx, ones_v, sem):
        c = lax.axis_index("c")
        s = lax.axis_index("s")
        wid = c * NS + s
        pltpu.sync_copy(zeros_hbm.at[pl.ds(s * rows_per_tile, rows_per_tile)],
                        acc.at[pl.ds(s * rows_per_tile, rows_per_tile)])
        pltpu.sync_copy(ones_hbm, ones_v)
        plsc.subcore_barrier()
        for it in range(n_it):
            row_base = wid * (per_tile // 128) + it * KJ
            pltpu.sync_copy(dst_hbm.at[pl.ds(row_base, KJ)], didx)

            def body(j, _):
                pltpu.sync_copy(ones_v.at[pl.ds(j * 128, 128)],
                                acc.at[didx.at[j]], add=True)
                return 0

            lax.fori_loop(0, KJ, body, 0)
        plsc.subcore_barrier()
        pltpu.sync_copy(acc.at[pl.ds(s * rows_per_tile, rows_per_tile)],
                        out_hbm.at[c, pl.ds(s * rows_per_tile, rows_per_tile)])

    return deg_kernel


def _make_agg_kernel(n_pad, e_pad):
    per_tile = e_pad // NW
    n_it = per_tile // CHUNK
    rows_per_tile = n_pad // NS

    @functools.partial(
        pl.kernel,
        out_type=jax.ShapeDtypeStruct((NC, n_pad, F), jnp.float32),
        mesh=_sc_mesh(),
        scratch_types=[
            pltpu.VMEM_SHARED((n_pad, F), jnp.float32),
            pltpu.VMEM((KJ, 128), jnp.int32),
            pltpu.VMEM((KJ, 128), jnp.int32),
            pltpu.VMEM((CHUNK, F), jnp.float32),
            pltpu.SemaphoreType.DMA,
        ],
    )
    def agg_kernel(src_hbm, dst_hbm, z_hbm, zeros_hbm, out_hbm,
                   acc, sidx, didx, rows, sem):
        c = lax.axis_index("c")
        s = lax.axis_index("s")
        wid = c * NS + s
        pltpu.sync_copy(zeros_hbm.at[pl.ds(s * rows_per_tile, rows_per_tile)],
                        acc.at[pl.ds(s * rows_per_tile, rows_per_tile)])
        plsc.subcore_barrier()
        for it in range(n_it):
            row_base = wid * (per_tile // 128) + it * KJ
            pltpu.sync_copy(src_hbm.at[pl.ds(row_base, KJ)], sidx)
            pltpu.sync_copy(dst_hbm.at[pl.ds(row_base, KJ)], didx)

            def gbody(j, _):
                pltpu.async_copy(z_hbm.at[sidx.at[j]],
                                 rows.at[pl.ds(j * 128, 128)], sem).wait()
                return 0

            lax.fori_loop(0, KJ, gbody, 0)

            def sbody(j, _):
                pltpu.sync_copy(rows.at[pl.ds(j * 128, 128)],
                                acc.at[didx.at[j]], add=True)
                return 0

            lax.fori_loop(0, KJ, sbody, 0)
        plsc.subcore_barrier()
        pltpu.sync_copy(acc.at[pl.ds(s * rows_per_tile, rows_per_tile)],
                        out_hbm.at[c, pl.ds(s * rows_per_tile, rows_per_tile)])

    return agg_kernel


def _tc1(x_ref, w1_ref, d0_ref, d1_ref, z1_ref, dis_ref):
    deg = d0_ref[...] + d1_ref[...] + 1.0
    dis = lax.rsqrt(deg)
    xw = jnp.dot(x_ref[...], w1_ref[...], preferred_element_type=jnp.float32)
    z1_ref[...] = dis * xw
    dis_ref[...] = jnp.broadcast_to(dis, dis_ref.shape)


def _tc2(p0_ref, p1_ref, z1_ref, dis_ref, mask_ref, w2_ref, b1_ref, z2_ref):
    out1 = dis_ref[...] * (p0_ref[...] + p1_ref[...] + z1_ref[...]) + b1_ref[...]
    h1 = mask_ref[...] * jnp.maximum(out1, 0.0)
    z2_ref[...] = dis_ref[...] * jnp.dot(h1, w2_ref[...],
                                         preferred_element_type=jnp.float32)


def _tc3(q0_ref, q1_ref, z2_ref, dis_ref, b2_ref, h2_ref):
    out2 = dis_ref[...] * (q0_ref[...] + q1_ref[...] + z2_ref[...]) + b2_ref[...]
    h2_ref[...] = jnp.maximum(out2, 0.0)


def _tc4(t_ref, f1_ref, b1_ref, f2_ref, b2_ref, o_ref):
    r = jnp.maximum(jnp.dot(t_ref[...], f1_ref[...],
                            preferred_element_type=jnp.float32) + b1_ref[...], 0.0)
    o_ref[...] = jnp.dot(r, f2_ref[...],
                         preferred_element_type=jnp.float32) + b2_ref[...]


def kernel(x, edge_index, y, W1, b1, W2, b2, fc1_w, fc1_b, fc2_w, fc2_b):
    n, d = x.shape
    e = edge_index.shape[1]
    yn = y.shape[0]

    n_pad = ((n + NS * 16 - 1) // (NS * 16)) * (NS * 16)
    n_pad = ((n_pad + 127) // 128) * 128
    per_tile = ((e + NW * CHUNK - 1) // (NW * CHUNK)) * CHUNK
    e_pad = per_tile * NW

    xp = jnp.pad(x, ((0, n_pad - n), (0, 0)))
    pad_e = e_pad - e
    fill = jnp.full((pad_e,), n, dtype=jnp.int32)
    srcp = jnp.concatenate([edge_index[0], fill]).reshape(e_pad // 128, 128)
    dstp = jnp.concatenate([edge_index[1], fill]).reshape(e_pad // 128, 128)
    ones_e = jnp.ones((CHUNK,), jnp.float32)
    zeros_n = jnp.zeros((n_pad,), jnp.float32)
    zeros_nf = jnp.zeros((n_pad, F), jnp.float32)
    keep = jax.random.bernoulli(jax.random.key(42), 0.6, (n, F))
    mask = jnp.pad(jnp.where(keep, jnp.float32(1.0 / 0.6), jnp.float32(0.0)),
                   ((0, n_pad - n), (0, 0)))

    deg_kernel = _make_deg_kernel(n_pad, e_pad)
    agg_kernel = _make_agg_kernel(n_pad, e_pad)

    degp = deg_kernel(dstp, ones_e, zeros_n)

    z1, dis16 = pl.pallas_call(
        _tc1,
        out_shape=[jax.ShapeDtypeStruct((n_pad, F), jnp.float32),
                   jax.ShapeDtypeStruct((n_pad, F), jnp.float32)],
    )(xp, W1, degp[0].reshape(n_pad, 1), degp[1].reshape(n_pad, 1))

    p = agg_kernel(srcp, dstp, z1, zeros_nf)

    z2 = pl.pallas_call(
        _tc2,
        out_shape=jax.ShapeDtypeStruct((n_pad, F), jnp.float32),
    )(p[0], p[1], z1, dis16, mask, W2, b1.reshape(1, F))

    q = agg_kernel(srcp, dstp, z2, zeros_nf)

    h2 = pl.pallas_call(
        _tc3,
        out_shape=jax.ShapeDtypeStruct((n_pad, F), jnp.float32),
    )(q[0], q[1], z2, dis16, b2.reshape(1, F))

    idx0 = 1423
    step = 1431
    t = jnp.stack([h2[idx0 + step * k] for k in range(yn)])
    t8 = jnp.zeros((8, 128), jnp.float32).at[:yn, :F].set(t)
    f1p = jnp.zeros((128, 128), jnp.float32).at[:F, :fc1_w.shape[1]].set(fc1_w)
    b1p = jnp.zeros((1, 128), jnp.float32).at[0, :fc1_b.shape[0]].set(fc1_b)
    f2p = jnp.zeros((128, 128), jnp.float32).at[:fc2_w.shape[0], :1].set(fc2_w)
    b2p = jnp.zeros((1, 128), jnp.float32).at[0, 0].set(fc2_b[0])
    o = pl.pallas_call(
        _tc4,
        out_shape=jax.ShapeDtypeStruct((8, 128), jnp.float32),
    )(t8, f1p, b1p, f2p, b2p)
    return o[:yn, :1]


# speedup vs baseline: 51.1205x; 51.1205x over previous
import functools

import jax
import jax.numpy as jnp
from jax import lax
from jax.experimental import pallas as pl
from jax.experimental.pallas import tpu as pltpu
from jax.experimental.pallas import tpu_sc as plsc

NC = 2
NS = 16
NW = NC * NS

F = 16
CHUNK = 2048
KJ = CHUNK // 128


def _sc_mesh():
    return plsc.VectorSubcoreMesh(core_axis_name="c", subcore_axis_name="s")


_SC_PARAMS = pltpu.CompilerParams(use_tc_tiling_on_sc=False)


def _make_deg_kernel(n_pad, e_pad):
    per_tile = e_pad // NW
    n_it = per_tile // CHUNK
    rows_per_tile = n_pad // NS

    @functools.partial(
        pl.kernel,
        out_type=jax.ShapeDtypeStruct((NC, n_pad), jnp.float32),
        mesh=_sc_mesh(),
        compiler_params=_SC_PARAMS,
        scratch_types=[
            pltpu.VMEM_SHARED((n_pad,), jnp.float32),
            pltpu.VMEM((KJ, 128), jnp.int32),
            pltpu.VMEM((CHUNK,), jnp.float32),
            pltpu.SemaphoreType.DMA,
        ],
    )
    def deg_kernel(dst_hbm, ones_hbm, zeros_hbm, out_hbm, acc, didx, ones_v, sem):
        c = lax.axis_index("c")
        s = lax.axis_index("s")
        wid = c * NS + s
        pltpu.sync_copy(zeros_hbm.at[pl.ds(s * rows_per_tile, rows_per_tile)],
                        acc.at[pl.ds(s * rows_per_tile, rows_per_tile)])
        pltpu.sync_copy(ones_hbm, ones_v)
        plsc.subcore_barrier()
        for it in range(n_it):
            row_base = wid * (per_tile // 128) + it * KJ
            pltpu.sync_copy(dst_hbm.at[pl.ds(row_base, KJ)], didx)

            def body(j, _):
                pltpu.sync_copy(ones_v.at[pl.ds(j * 128, 128)],
                                acc.at[didx.at[j]], add=True)
                return 0

            lax.fori_loop(0, KJ, body, 0)
        plsc.subcore_barrier()
        pltpu.sync_copy(acc.at[pl.ds(s * rows_per_tile, rows_per_tile)],
                        out_hbm.at[c, pl.ds(s * rows_per_tile, rows_per_tile)])

    return deg_kernel


def _make_agg_kernel(n_pad, e_pad):
    per_tile = e_pad // NW
    n_it = per_tile // CHUNK
    rows_per_tile = n_pad // NS

    @functools.partial(
        pl.kernel,
        out_type=jax.ShapeDtypeStruct((NC, n_pad, F), jnp.float32),
        mesh=_sc_mesh(),
        compiler_params=_SC_PARAMS,
        scratch_types=[
            pltpu.VMEM_SHARED((n_pad, F), jnp.float32),
            pltpu.VMEM_SHARED((n_pad, F), jnp.float32),
            pltpu.VMEM((KJ, 128), jnp.int32),
            pltpu.VMEM((KJ, 128), jnp.int32),
            pltpu.VMEM((CHUNK, F), jnp.float32),
            pltpu.SemaphoreType.DMA,
        ],
    )
    def agg_kernel(src_hbm, dst_hbm, z_hbm, zeros_hbm, out_hbm,
                   acc, z_sh, sidx, didx, rows, sem):
        c = lax.axis_index("c")
        s = lax.axis_index("s")
        wid = c * NS + s
        pltpu.sync_copy(zeros_hbm.at[pl.ds(s * rows_per_tile, rows_per_tile)],
                        acc.at[pl.ds(s * rows_per_tile, rows_per_tile)])
        pltpu.sync_copy(z_hbm.at[pl.ds(s * rows_per_tile, rows_per_tile)],
                        z_sh.at[pl.ds(s * rows_per_tile, rows_per_tile)])
        plsc.subcore_barrier()
        for it in range(n_it):
            row_base = wid * (per_tile // 128) + it * KJ
            pltpu.sync_copy(src_hbm.at[pl.ds(row_base, KJ)], sidx)
            pltpu.sync_copy(dst_hbm.at[pl.ds(row_base, KJ)], didx)

            def gbody(j, _):
                pltpu.async_copy(z_sh.at[sidx.at[j]],
                                 rows.at[pl.ds(j * 128, 128)], sem).wait()
                return 0

            lax.fori_loop(0, KJ, gbody, 0)

            def sbody(j, _):
                pltpu.sync_copy(rows.at[pl.ds(j * 128, 128)],
                                acc.at[didx.at[j]], add=True)
                return 0

            lax.fori_loop(0, KJ, sbody, 0)
        plsc.subcore_barrier()
        pltpu.sync_copy(acc.at[pl.ds(s * rows_per_tile, rows_per_tile)],
                        out_hbm.at[c, pl.ds(s * rows_per_tile, rows_per_tile)])

    return agg_kernel


def _tc1(x_ref, w1_ref, d0_ref, d1_ref, z1_ref, dis_ref):
    deg = d0_ref[...] + d1_ref[...] + 1.0
    dis = lax.rsqrt(deg)
    xw = jnp.dot(x_ref[...], w1_ref[...], preferred_element_type=jnp.float32)
    z1_ref[...] = dis * xw
    dis_ref[...] = jnp.broadcast_to(dis, dis_ref.shape)


def _tc2(p0_ref, p1_ref, z1_ref, dis_ref, mask_ref, w2_ref, b1_ref, z2_ref):
    out1 = dis_ref[...] * (p0_ref[...] + p1_ref[...] + z1_ref[...]) + b1_ref[...]
    h1 = mask_ref[...] * jnp.maximum(out1, 0.0)
    z2_ref[...] = dis_ref[...] * jnp.dot(h1, w2_ref[...],
                                         preferred_element_type=jnp.float32)


def _tc3(q0_ref, q1_ref, z2_ref, dis_ref, b2_ref, h2_ref):
    out2 = dis_ref[...] * (q0_ref[...] + q1_ref[...] + z2_ref[...]) + b2_ref[...]
    h2_ref[...] = jnp.maximum(out2, 0.0)


def _tc4(t_ref, f1_ref, b1_ref, f2_ref, b2_ref, o_ref):
    r = jnp.maximum(jnp.dot(t_ref[...], f1_ref[...],
                            preferred_element_type=jnp.float32) + b1_ref[...], 0.0)
    o_ref[...] = jnp.dot(r, f2_ref[...],
                         preferred_element_type=jnp.float32) + b2_ref[...]


def kernel(x, edge_index, y, W1, b1, W2, b2, fc1_w, fc1_b, fc2_w, fc2_b):
    n, d = x.shape
    e = edge_index.shape[1]
    yn = y.shape[0]

    n_pad = ((n + NS * 16 - 1) // (NS * 16)) * (NS * 16)
    n_pad = ((n_pad + 127) // 128) * 128
    per_tile = ((e + NW * CHUNK - 1) // (NW * CHUNK)) * CHUNK
    e_pad = per_tile * NW

    xp = jnp.pad(x, ((0, n_pad - n), (0, 0)))
    pad_e = e_pad - e
    fill = jnp.full((pad_e,), n, dtype=jnp.int32)
    srcp = jnp.concatenate([edge_index[0], fill]).reshape(e_pad // 128, 128)
    dstp = jnp.concatenate([edge_index[1], fill]).reshape(e_pad // 128, 128)
    ones_e = jnp.ones((CHUNK,), jnp.float32)
    zeros_n = jnp.zeros((n_pad,), jnp.float32)
    zeros_nf = jnp.zeros((n_pad, F), jnp.float32)
    keep = jax.random.bernoulli(jax.random.key(42), 0.6, (n, F))
    mask = jnp.pad(jnp.where(keep, jnp.float32(1.0 / 0.6), jnp.float32(0.0)),
                   ((0, n_pad - n), (0, 0)))

    deg_kernel = _make_deg_kernel(n_pad, e_pad)
    agg_kernel = _make_agg_kernel(n_pad, e_pad)

    degp = deg_kernel(dstp, ones_e, zeros_n)

    z1, dis16 = pl.pallas_call(
        _tc1,
        out_shape=[jax.ShapeDtypeStruct((n_pad, F), jnp.float32),
                   jax.ShapeDtypeStruct((n_pad, F), jnp.float32)],
    )(xp, W1, degp[0].reshape(n_pad, 1), degp[1].reshape(n_pad, 1))

    p = agg_kernel(srcp, dstp, z1, zeros_nf)

    z2 = pl.pallas_call(
        _tc2,
        out_shape=jax.ShapeDtypeStruct((n_pad, F), jnp.float32),
    )(p[0], p[1], z1, dis16, mask, W2, b1.reshape(1, F))

    q = agg_kernel(srcp, dstp, z2, zeros_nf)

    h2 = pl.pallas_call(
        _tc3,
        out_shape=jax.ShapeDtypeStruct((n_pad, F), jnp.float32),
    )(q[0], q[1], z2, dis16, b2.reshape(1, F))

    idx0 = 1423
    step = 1431
    t = jnp.stack([h2[idx0 + step * k] for k in range(yn)])
    t8 = jnp.zeros((8, 128), jnp.float32).at[:yn, :F].set(t)
    f1p = jnp.zeros((128, 128), jnp.float32).at[:F, :fc1_w.shape[1]].set(fc1_w)
    b1p = jnp.zeros((1, 128), jnp.float32).at[0, :fc1_b.shape[0]].set(fc1_b)
    f2p = jnp.zeros((128, 128), jnp.float32).at[:fc2_w.shape[0], :1].set(fc2_w)
    b2p = jnp.zeros((1, 128), jnp.float32).at[0, 0].set(fc2_b[0])
    o = pl.pallas_call(
        _tc4,
        out_shape=jax.ShapeDtypeStruct((8, 128), jnp.float32),
    )(t8, f1p, b1p, f2p, b2p)
    return o[:yn, :1]


# speedup vs baseline: 58.7900x; 1.1500x over previous
import functools

import jax
import jax.numpy as jnp
from jax import lax
from jax.experimental import pallas as pl
from jax.experimental.pallas import tpu as pltpu
from jax.experimental.pallas import tpu_sc as plsc

NC = 2
NS = 16
NW = NC * NS

F = 16
CHUNK = 2048
KJ = CHUNK // 128


def _sc_mesh():
    return plsc.VectorSubcoreMesh(core_axis_name="c", subcore_axis_name="s")


_SC_PARAMS = pltpu.CompilerParams(use_tc_tiling_on_sc=False)


def _make_deg_kernel(n_pad, e_pad):
    per_tile = e_pad // NW
    n_it = per_tile // CHUNK
    rows_per_tile = n_pad // NS

    @functools.partial(
        pl.kernel,
        out_type=jax.ShapeDtypeStruct((NC, n_pad, F), jnp.float32),
        mesh=_sc_mesh(),
        compiler_params=_SC_PARAMS,
        scratch_types=[
            pltpu.VMEM_SHARED((n_pad,), jnp.float32),
            pltpu.VMEM((KJ, 128), jnp.int32),
            pltpu.VMEM((CHUNK,), jnp.float32),
            pltpu.VMEM((rows_per_tile,), jnp.float32),
            pltpu.VMEM((rows_per_tile, F), jnp.float32),
            pltpu.SemaphoreType.DMA,
        ],
    )
    def deg_kernel(dst_hbm, ones_hbm, zeros_hbm, out_hbm, acc, didx, ones_v,
                   db, d16, sem):
        c = lax.axis_index("c")
        s = lax.axis_index("s")
        wid = c * NS + s
        pltpu.sync_copy(zeros_hbm.at[pl.ds(s * rows_per_tile, rows_per_tile)],
                        acc.at[pl.ds(s * rows_per_tile, rows_per_tile)])
        pltpu.sync_copy(ones_hbm, ones_v)
        plsc.subcore_barrier()
        for it in range(n_it):
            row_base = wid * (per_tile // 128) + it * KJ
            pltpu.sync_copy(dst_hbm.at[pl.ds(row_base, KJ)], didx)

            def body(j, _):
                pltpu.sync_copy(ones_v.at[pl.ds(j * 128, 128)],
                                acc.at[didx.at[j]], add=True)
                return 0

            lax.fori_loop(0, KJ, body, 0)
        plsc.subcore_barrier()
        pltpu.sync_copy(acc.at[pl.ds(s * rows_per_tile, rows_per_tile)], db)

        def bbody(i, _):
            v = db[pl.ds(i * 16, 16)]
            for j in range(16):
                d16[i * 16 + j, :] = jnp.full((F,), v[j], dtype=jnp.float32)
            return 0

        lax.fori_loop(0, rows_per_tile // 16, bbody, 0)
        pltpu.sync_copy(d16, out_hbm.at[c, pl.ds(s * rows_per_tile, rows_per_tile)])

    return deg_kernel


def _make_agg_kernel(n_pad, e_pad, out_rows=None):
    per_tile = e_pad // NW
    n_it = per_tile // CHUNK
    rows_per_tile = n_pad // NS
    out_shape = ((NC, n_pad, F) if out_rows is None
                 else (NC, 8, F))

    @functools.partial(
        pl.kernel,
        out_type=jax.ShapeDtypeStruct(out_shape, jnp.float32),
        mesh=_sc_mesh(),
        compiler_params=_SC_PARAMS,
        scratch_types=[
            pltpu.VMEM_SHARED((n_pad, F), jnp.float32),
            pltpu.VMEM_SHARED((n_pad, F), jnp.float32),
            pltpu.VMEM((KJ, 128), jnp.int32),
            pltpu.VMEM((KJ, 128), jnp.int32),
            pltpu.VMEM((CHUNK, F), jnp.float32),
            pltpu.SemaphoreType.DMA,
        ],
    )
    def agg_kernel(src_hbm, dst_hbm, z_hbm, zeros_hbm, out_hbm,
                   acc, z_sh, sidx, didx, rows, sem):
        c = lax.axis_index("c")
        s = lax.axis_index("s")
        wid = c * NS + s
        pltpu.sync_copy(zeros_hbm.at[pl.ds(s * rows_per_tile, rows_per_tile)],
                        acc.at[pl.ds(s * rows_per_tile, rows_per_tile)])
        pltpu.sync_copy(z_hbm.at[pl.ds(s * rows_per_tile, rows_per_tile)],
                        z_sh.at[pl.ds(s * rows_per_tile, rows_per_tile)])
        plsc.subcore_barrier()
        for it in range(n_it):
            row_base = wid * (per_tile // 128) + it * KJ
            pltpu.sync_copy(src_hbm.at[pl.ds(row_base, KJ)], sidx)
            pltpu.sync_copy(dst_hbm.at[pl.ds(row_base, KJ)], didx)

            def gbody(j, _):
                pltpu.async_copy(z_sh.at[sidx.at[j]],
                                 rows.at[pl.ds(j * 128, 128)], sem).wait()
                return 0

            lax.fori_loop(0, KJ, gbody, 0)

            def sbody(j, _):
                pltpu.sync_copy(rows.at[pl.ds(j * 128, 128)],
                                acc.at[didx.at[j]], add=True)
                return 0

            lax.fori_loop(0, KJ, sbody, 0)
        plsc.subcore_barrier()
        if out_rows is None:
            pltpu.sync_copy(acc.at[pl.ds(s * rows_per_tile, rows_per_tile)],
                            out_hbm.at[c, pl.ds(s * rows_per_tile, rows_per_tile)])
        else:
            @pl.when(s == 0)
            def _():
                for k, r in enumerate(out_rows):
                    pltpu.sync_copy(acc.at[pl.ds(r, 1)],
                                    out_hbm.at[c, pl.ds(k, 1)])

    return agg_kernel


def _rows8(v, rows):
    return jnp.concatenate(
        [v[r:r + 1] for r in rows] + [jnp.zeros((8 - len(rows), v.shape[1]),
                                                jnp.float32)], axis=0)


def _make_tc1(rows):
    def _tc1(x_ref, w1_ref, d_ref, z1_ref, dis_ref, disr_ref):
        deg = d_ref[0] + d_ref[1] + 1.0
        dis = lax.rsqrt(deg)
        xw = jnp.dot(x_ref[...], w1_ref[...], preferred_element_type=jnp.float32)
        z1_ref[...] = dis * xw
        dis_ref[...] = dis
        disr_ref[...] = _rows8(dis, rows)
    return _tc1


def _make_tc2(rows):
    def _tc2(p_ref, z1_ref, dis_ref, mask_ref, w2_ref, b1_ref,
             z2_ref, z2r_ref):
        out1 = dis_ref[...] * (p_ref[0] + p_ref[1] + z1_ref[...]) + b1_ref[...]
        h1 = mask_ref[...] * jnp.maximum(out1, 0.0)
        z2 = dis_ref[...] * jnp.dot(h1, w2_ref[...],
                                    preferred_element_type=jnp.float32)
        z2_ref[...] = z2
        z2r_ref[...] = _rows8(z2, rows)
    return _tc2


def _tc3(qr_ref, z2r_ref, disr_ref, b2_ref, f1_ref, b1_ref, f2_ref, b2m_ref,
         o_ref):
    out2 = disr_ref[...] * (qr_ref[0] + qr_ref[1] + z2r_ref[...]) + b2_ref[...]
    t = jnp.maximum(out2, 0.0)
    r = jnp.maximum(jnp.dot(t, f1_ref[...],
                            preferred_element_type=jnp.float32) + b1_ref[...], 0.0)
    o_ref[...] = jnp.dot(r, f2_ref[...],
                         preferred_element_type=jnp.float32) + b2m_ref[...]


def kernel(x, edge_index, y, W1, b1, W2, b2, fc1_w, fc1_b, fc2_w, fc2_b):
    n, d = x.shape
    e = edge_index.shape[1]
    yn = y.shape[0]

    n_pad = ((n + NS * 16 - 1) // (NS * 16)) * (NS * 16)
    n_pad = ((n_pad + 127) // 128) * 128
    per_tile = ((e + NW * CHUNK - 1) // (NW * CHUNK)) * CHUNK
    e_pad = per_tile * NW

    xp = jnp.pad(x, ((0, n_pad - n), (0, 0)))
    pad_e = e_pad - e
    fill = jnp.full((pad_e,), n, dtype=jnp.int32)
    srcp = jnp.concatenate([edge_index[0], fill]).reshape(e_pad // 128, 128)
    dstp = jnp.concatenate([edge_index[1], fill]).reshape(e_pad // 128, 128)
    ones_e = jnp.ones((CHUNK,), jnp.float32)
    zeros_n = jnp.zeros((n_pad,), jnp.float32)
    zeros_nf = jnp.zeros((n_pad, F), jnp.float32)
    keep = jax.random.bernoulli(jax.random.key(42), 0.6, (n, F))
    mask = jnp.pad(jnp.where(keep, jnp.float32(1.0 / 0.6), jnp.float32(0.0)),
                   ((0, n_pad - n), (0, 0)))

    rows = tuple(1423 + 1431 * k for k in range(yn))
    deg_kernel = _make_deg_kernel(n_pad, e_pad)
    agg_kernel = _make_agg_kernel(n_pad, e_pad)
    agg2_kernel = _make_agg_kernel(n_pad, e_pad, out_rows=rows)

    deg16 = deg_kernel(dstp, ones_e, zeros_n)

    z1, dis16, disr = pl.pallas_call(
        _make_tc1(rows),
        out_shape=[jax.ShapeDtypeStruct((n_pad, F), jnp.float32),
                   jax.ShapeDtypeStruct((n_pad, F), jnp.float32),
                   jax.ShapeDtypeStruct((8, F), jnp.float32)],
    )(xp, W1, deg16)

    p = agg_kernel(srcp, dstp, z1, zeros_nf)

    z2, z2r = pl.pallas_call(
        _make_tc2(rows),
        out_shape=[jax.ShapeDtypeStruct((n_pad, F), jnp.float32),
                   jax.ShapeDtypeStruct((8, F), jnp.float32)],
    )(p, z1, dis16, mask, W2, b1.reshape(1, F))

    qr = agg2_kernel(srcp, dstp, z2, zeros_nf)

    f1b = jnp.broadcast_to(fc1_b.reshape(1, -1), (1, fc1_w.shape[1]))
    o = pl.pallas_call(
        _tc3,
        out_shape=jax.ShapeDtypeStruct((8, 1), jnp.float32),
    )(qr, z2r, disr, b2.reshape(1, F), fc1_w, f1b, fc2_w,
      fc2_b.reshape(1, 1))
    return o[:yn]


# speedup vs baseline: 64.5291x; 1.0976x over previous
import functools

import jax
import jax.numpy as jnp
from jax import lax
from jax.experimental import pallas as pl
from jax.experimental.pallas import tpu as pltpu
from jax.experimental.pallas import tpu_sc as plsc

NC = 2
NS = 16
NW = NC * NS

F = 16
CHUNK = 2048
KJ = CHUNK // 128


def _sc_mesh():
    return plsc.VectorSubcoreMesh(core_axis_name="c", subcore_axis_name="s")


_SC_PARAMS = pltpu.CompilerParams(use_tc_tiling_on_sc=False)


def _make_deg_kernel(n_pad, e_pad):
    per_tile = e_pad // NW
    n_it = per_tile // CHUNK
    rows_per_tile = n_pad // NS

    @functools.partial(
        pl.kernel,
        out_type=jax.ShapeDtypeStruct((NC, n_pad, F), jnp.float32),
        mesh=_sc_mesh(),
        compiler_params=_SC_PARAMS,
        scratch_types=[
            pltpu.VMEM_SHARED((n_pad,), jnp.float32),
            pltpu.VMEM((KJ, 128), jnp.int32),
            pltpu.VMEM((CHUNK,), jnp.float32),
            pltpu.VMEM((rows_per_tile,), jnp.float32),
            pltpu.VMEM((rows_per_tile, F), jnp.float32),
            pltpu.SemaphoreType.DMA,
        ],
    )
    def deg_kernel(dst_hbm, ones_hbm, zeros_hbm, out_hbm, acc, didx, ones_v,
                   db, d16, sem):
        c = lax.axis_index("c")
        s = lax.axis_index("s")
        wid = c * NS + s
        pltpu.sync_copy(zeros_hbm.at[pl.ds(s * rows_per_tile, rows_per_tile)],
                        acc.at[pl.ds(s * rows_per_tile, rows_per_tile)])
        pltpu.sync_copy(ones_hbm, ones_v)
        plsc.subcore_barrier()
        for it in range(n_it):
            row_base = wid * (per_tile // 128) + it * KJ
            pltpu.sync_copy(dst_hbm.at[pl.ds(row_base, KJ)], didx)

            def body(j, _):
                pltpu.async_copy(ones_v.at[pl.ds(j * 128, 128)],
                                 acc.at[didx.at[j]], sem, add=True)
                return 0

            lax.fori_loop(0, KJ, body, 0)
            pltpu.make_async_copy(ones_hbm, ones_v, sem).wait()
        plsc.subcore_barrier()
        pltpu.sync_copy(acc.at[pl.ds(s * rows_per_tile, rows_per_tile)], db)

        def bbody(i, _):
            v = db[pl.ds(i * 16, 16)]
            for j in range(16):
                d16[i * 16 + j, :] = jnp.full((F,), v[j], dtype=jnp.float32)
            return 0

        lax.fori_loop(0, rows_per_tile // 16, bbody, 0)
        pltpu.sync_copy(d16, out_hbm.at[c, pl.ds(s * rows_per_tile, rows_per_tile)])

    return deg_kernel


def _make_agg_kernel(n_pad, e_pad, out_rows=None):
    per_tile = e_pad // NW
    n_it = per_tile // CHUNK
    rows_per_tile = n_pad // NS
    out_shape = ((NC, n_pad, F) if out_rows is None
                 else (NC, 8, F))

    @functools.partial(
        pl.kernel,
        out_type=jax.ShapeDtypeStruct(out_shape, jnp.float32),
        mesh=_sc_mesh(),
        compiler_params=_SC_PARAMS,
        scratch_types=[
            pltpu.VMEM_SHARED((n_pad, F), jnp.float32),
            pltpu.VMEM_SHARED((n_pad, F), jnp.float32),
            pltpu.VMEM((KJ, 128), jnp.int32),
            pltpu.VMEM((KJ, 128), jnp.int32),
            pltpu.VMEM((CHUNK, F), jnp.float32),
            pltpu.SemaphoreType.DMA,
        ],
    )
    def agg_kernel(src_hbm, dst_hbm, z_hbm, zeros_hbm, out_hbm,
                   acc, z_sh, sidx, didx, rows, sem):
        c = lax.axis_index("c")
        s = lax.axis_index("s")
        wid = c * NS + s
        pltpu.sync_copy(zeros_hbm.at[pl.ds(s * rows_per_tile, rows_per_tile)],
                        acc.at[pl.ds(s * rows_per_tile, rows_per_tile)])
        pltpu.sync_copy(z_hbm.at[pl.ds(s * rows_per_tile, rows_per_tile)],
                        z_sh.at[pl.ds(s * rows_per_tile, rows_per_tile)])
        plsc.subcore_barrier()
        for it in range(n_it):
            row_base = wid * (per_tile // 128) + it * KJ
            pltpu.sync_copy(src_hbm.at[pl.ds(row_base, KJ)], sidx)
            pltpu.sync_copy(dst_hbm.at[pl.ds(row_base, KJ)], didx)

            def gbody(j, _):
                pltpu.async_copy(z_sh.at[sidx.at[j]],
                                 rows.at[pl.ds(j * 128, 128)], sem)
                return 0

            lax.fori_loop(0, KJ, gbody, 0)
            pltpu.make_async_copy(zeros_hbm.at[pl.ds(0, CHUNK)], rows,
                                  sem).wait()

            def sbody(j, _):
                pltpu.async_copy(rows.at[pl.ds(j * 128, 128)],
                                 acc.at[didx.at[j]], sem, add=True)
                return 0

            lax.fori_loop(0, KJ, sbody, 0)
            pltpu.make_async_copy(zeros_hbm.at[pl.ds(0, CHUNK)], rows,
                                  sem).wait()
        plsc.subcore_barrier()
        if out_rows is None:
            pltpu.sync_copy(acc.at[pl.ds(s * rows_per_tile, rows_per_tile)],
                            out_hbm.at[c, pl.ds(s * rows_per_tile, rows_per_tile)])
        else:
            @pl.when(s == 0)
            def _():
                for k, r in enumerate(out_rows):
                    pltpu.sync_copy(acc.at[pl.ds(r, 1)],
                                    out_hbm.at[c, pl.ds(k, 1)])

    return agg_kernel


def _rows8(v, rows):
    return jnp.concatenate(
        [v[r:r + 1] for r in rows] + [jnp.zeros((8 - len(rows), v.shape[1]),
                                                jnp.float32)], axis=0)


def _make_tc1(rows):
    def _tc1(x_ref, w1_ref, d_ref, z1_ref, dis_ref, disr_ref):
        deg = d_ref[0] + d_ref[1] + 1.0
        dis = lax.rsqrt(deg)
        xw = jnp.dot(x_ref[...], w1_ref[...], preferred_element_type=jnp.float32)
        z1_ref[...] = dis * xw
        dis_ref[...] = dis
        disr_ref[...] = _rows8(dis, rows)
    return _tc1


def _make_tc2(rows):
    def _tc2(p_ref, z1_ref, dis_ref, mask_ref, w2_ref, b1_ref,
             z2_ref, z2r_ref):
        out1 = dis_ref[...] * (p_ref[0] + p_ref[1] + z1_ref[...]) + b1_ref[...]
        h1 = mask_ref[...] * jnp.maximum(out1, 0.0)
        z2 = dis_ref[...] * jnp.dot(h1, w2_ref[...],
                                    preferred_element_type=jnp.float32)
        z2_ref[...] = z2
        z2r_ref[...] = _rows8(z2, rows)
    return _tc2


def _tc3(qr_ref, z2r_ref, disr_ref, b2_ref, f1_ref, b1_ref, f2_ref, b2m_ref,
         o_ref):
    out2 = disr_ref[...] * (qr_ref[0] + qr_ref[1] + z2r_ref[...]) + b2_ref[...]
    t = jnp.maximum(out2, 0.0)
    r = jnp.maximum(jnp.dot(t, f1_ref[...],
                            preferred_element_type=jnp.float32) + b1_ref[...], 0.0)
    o_ref[...] = jnp.dot(r, f2_ref[...],
                         preferred_element_type=jnp.float32) + b2m_ref[...]


def kernel(x, edge_index, y, W1, b1, W2, b2, fc1_w, fc1_b, fc2_w, fc2_b):
    n, d = x.shape
    e = edge_index.shape[1]
    yn = y.shape[0]

    n_pad = ((n + NS * 16 - 1) // (NS * 16)) * (NS * 16)
    n_pad = ((n_pad + 127) // 128) * 128
    per_tile = ((e + NW * CHUNK - 1) // (NW * CHUNK)) * CHUNK
    e_pad = per_tile * NW

    xp = jnp.pad(x, ((0, n_pad - n), (0, 0)))
    pad_e = e_pad - e
    fill = jnp.full((pad_e,), n, dtype=jnp.int32)
    srcp = jnp.concatenate([edge_index[0], fill]).reshape(e_pad // 128, 128)
    dstp = jnp.concatenate([edge_index[1], fill]).reshape(e_pad // 128, 128)
    ones_e = jnp.ones((CHUNK,), jnp.float32)
    zeros_n = jnp.zeros((n_pad,), jnp.float32)
    zeros_nf = jnp.zeros((n_pad, F), jnp.float32)
    keep = jax.random.bernoulli(jax.random.key(42), 0.6, (n, F))
    mask = jnp.pad(jnp.where(keep, jnp.float32(1.0 / 0.6), jnp.float32(0.0)),
                   ((0, n_pad - n), (0, 0)))

    rows = tuple(1423 + 1431 * k for k in range(yn))
    deg_kernel = _make_deg_kernel(n_pad, e_pad)
    agg_kernel = _make_agg_kernel(n_pad, e_pad)
    agg2_kernel = _make_agg_kernel(n_pad, e_pad, out_rows=rows)

    deg16 = deg_kernel(dstp, ones_e, zeros_n)

    z1, dis16, disr = pl.pallas_call(
        _make_tc1(rows),
        out_shape=[jax.ShapeDtypeStruct((n_pad, F), jnp.float32),
                   jax.ShapeDtypeStruct((n_pad, F), jnp.float32),
                   jax.ShapeDtypeStruct((8, F), jnp.float32)],
    )(xp, W1, deg16)

    p = agg_kernel(srcp, dstp, z1, zeros_nf)

    z2, z2r = pl.pallas_call(
        _make_tc2(rows),
        out_shape=[jax.ShapeDtypeStruct((n_pad, F), jnp.float32),
                   jax.ShapeDtypeStruct((8, F), jnp.float32)],
    )(p, z1, dis16, mask, W2, b1.reshape(1, F))

    qr = agg2_kernel(srcp, dstp, z2, zeros_nf)

    f1b = jnp.broadcast_to(fc1_b.reshape(1, -1), (1, fc1_w.shape[1]))
    o = pl.pallas_call(
        _tc3,
        out_shape=jax.ShapeDtypeStruct((8, 1), jnp.float32),
    )(qr, z2r, disr, b2.reshape(1, F), fc1_w, f1b, fc2_w,
      fc2_b.reshape(1, 1))
    return o[:yn]


# speedup vs baseline: 72.6707x; 1.1262x over previous
import functools

import jax
import jax.numpy as jnp
from jax import lax
from jax.experimental import pallas as pl
from jax.experimental.pallas import tpu as pltpu
from jax.experimental.pallas import tpu_sc as plsc

NC = 2
NS = 16
NW = NC * NS

F = 16
CHUNK = 2048
KJ = CHUNK // 128


def _sc_mesh():
    return plsc.VectorSubcoreMesh(core_axis_name="c", subcore_axis_name="s")


_SC_PARAMS = pltpu.CompilerParams(use_tc_tiling_on_sc=False)


def _make_deg_kernel(n_pad, e_pad):
    per_tile = e_pad // NW
    n_it = per_tile // CHUNK
    rows_per_tile = n_pad // NS

    @functools.partial(
        pl.kernel,
        out_type=jax.ShapeDtypeStruct((NC, n_pad, F), jnp.float32),
        mesh=_sc_mesh(),
        compiler_params=_SC_PARAMS,
        scratch_types=[
            pltpu.VMEM_SHARED((n_pad,), jnp.float32),
            pltpu.VMEM((KJ, 128), jnp.int32),
            pltpu.VMEM((KJ, 128), jnp.int32),
            pltpu.VMEM((CHUNK,), jnp.float32),
            pltpu.VMEM((rows_per_tile,), jnp.float32),
            pltpu.VMEM((rows_per_tile, F), jnp.float32),
            pltpu.SemaphoreType.DMA,
        ],
    )
    def deg_kernel(ed_hbm, ones_hbm, zeros_hbm, out_hbm, acc, didx0, didx1,
                   ones_v, db, d16, sem):
        c = lax.axis_index("c")
        s = lax.axis_index("s")
        wid = c * NS + s
        didx = (didx0, didx1)
        pltpu.sync_copy(zeros_hbm.at[pl.ds(s * rows_per_tile, rows_per_tile)],
                        acc.at[pl.ds(s * rows_per_tile, rows_per_tile)])
        pltpu.sync_copy(ones_hbm, ones_v)
        plsc.subcore_barrier()
        pltpu.sync_copy(ed_hbm.at[1, pl.ds(wid * (per_tile // 128), KJ)],
                        didx[0])
        for it in range(n_it):
            b = it & 1

            def body(j, _):
                pltpu.async_copy(ones_v.at[pl.ds(j * 128, 128)],
                                 acc.at[didx[b].at[j]], sem, add=True)
                return 0

            lax.fori_loop(0, KJ, body, 0)
            if it + 1 < n_it:
                row_base = wid * (per_tile // 128) + (it + 1) * KJ
                pltpu.sync_copy(ed_hbm.at[1, pl.ds(row_base, KJ)],
                                didx[1 - b])
            pltpu.make_async_copy(ones_hbm, ones_v, sem).wait()
        plsc.subcore_barrier()
        pltpu.sync_copy(acc.at[pl.ds(s * rows_per_tile, rows_per_tile)], db)

        def bbody(i, _):
            v = db[pl.ds(i * 16, 16)]
            for j in range(16):
                d16[i * 16 + j, :] = jnp.full((F,), v[j], dtype=jnp.float32)
            return 0

        lax.fori_loop(0, rows_per_tile // 16, bbody, 0)
        pltpu.sync_copy(d16, out_hbm.at[c, pl.ds(s * rows_per_tile,
                                                 rows_per_tile)])

    return deg_kernel


def _make_agg_kernel(n_pad, e_pad, out_rows=None):
    per_tile = e_pad // NW
    n_it = per_tile // CHUNK
    rows_per_tile = n_pad // NS
    out_shape = (NC, n_pad, F) if out_rows is None else (NC, 8, F)

    @functools.partial(
        pl.kernel,
        out_type=jax.ShapeDtypeStruct(out_shape, jnp.float32),
        mesh=_sc_mesh(),
        compiler_params=_SC_PARAMS,
        scratch_types=[
            pltpu.VMEM_SHARED((n_pad, F), jnp.float32),
            pltpu.VMEM_SHARED((n_pad, F), jnp.float32),
            pltpu.VMEM((KJ, 128), jnp.int32),
            pltpu.VMEM((KJ, 128), jnp.int32),
            pltpu.VMEM((KJ, 128), jnp.int32),
            pltpu.VMEM((KJ, 128), jnp.int32),
            pltpu.VMEM((CHUNK, F), jnp.float32),
            pltpu.VMEM((CHUNK, F), jnp.float32),
            pltpu.SemaphoreType.DMA,
            pltpu.SemaphoreType.DMA,
        ],
    )
    def agg_kernel(ed_hbm, z_hbm, zeros_hbm, out_hbm, acc, z_sh,
                   sidx0, sidx1, didx0, didx1, rows0, rows1, semg, sems):
        c = lax.axis_index("c")
        s = lax.axis_index("s")
        wid = c * NS + s
        sidx = (sidx0, sidx1)
        didx = (didx0, didx1)
        rows = (rows0, rows1)
        pltpu.sync_copy(zeros_hbm.at[pl.ds(s * rows_per_tile, rows_per_tile)],
                        acc.at[pl.ds(s * rows_per_tile, rows_per_tile)])
        pltpu.sync_copy(z_hbm.at[pl.ds(s * rows_per_tile, rows_per_tile)],
                        z_sh.at[pl.ds(s * rows_per_tile, rows_per_tile)])
        plsc.subcore_barrier()

        def stage(it, b):
            row_base = wid * (per_tile // 128) + it * KJ
            pltpu.sync_copy(ed_hbm.at[0, pl.ds(row_base, KJ)], sidx[b])
            pltpu.sync_copy(ed_hbm.at[1, pl.ds(row_base, KJ)], didx[b])

        def fire_gathers(b):
            def gbody(j, _):
                pltpu.async_copy(z_sh.at[sidx[b].at[j]],
                                 rows[b].at[pl.ds(j * 128, 128)], semg)
                return 0
            lax.fori_loop(0, KJ, gbody, 0)

        def drain(sem):
            pltpu.make_async_copy(zeros_hbm.at[pl.ds(0, CHUNK)], rows[0],
                                  sem).wait()

        stage(0, 0)
        fire_gathers(0)
        for it in range(n_it):
            b = it & 1
            drain(semg)

            def sbody(j, _):
                pltpu.async_copy(rows[b].at[pl.ds(j * 128, 128)],
                                 acc.at[didx[b].at[j]], sems, add=True)
                return 0

            lax.fori_loop(0, KJ, sbody, 0)
            if it + 1 < n_it:
                stage(it + 1, 1 - b)
                fire_gathers(1 - b)
            drain(sems)
        plsc.subcore_barrier()
        if out_rows is None:
            pltpu.sync_copy(acc.at[pl.ds(s * rows_per_tile, rows_per_tile)],
                            out_hbm.at[c, pl.ds(s * rows_per_tile,
                                                rows_per_tile)])
        else:
            @pl.when(s == 0)
            def _():
                for k, r in enumerate(out_rows):
                    pltpu.sync_copy(acc.at[pl.ds(r, 1)],
                                    out_hbm.at[c, pl.ds(k, 1)])

    return agg_kernel


def _rows8(v, rows):
    return jnp.concatenate(
        [v[r:r + 1] for r in rows] + [jnp.zeros((8 - len(rows), v.shape[1]),
                                                jnp.float32)], axis=0)


def _tc1a(n_pad, x_ref, w1_ref, xw_ref):
    xw = jnp.dot(x_ref[...], w1_ref[...], preferred_element_type=jnp.float32)
    xw_ref[...] = jnp.concatenate(
        [xw, jnp.zeros((n_pad - xw.shape[0], xw.shape[1]), jnp.float32)])


def _make_tc1(rows):
    def _tc1(xw_ref, d_ref, z1_ref, dis_ref, disr_ref):
        deg = d_ref[0] + d_ref[1] + 1.0
        dis = lax.rsqrt(deg)
        z1_ref[...] = dis * xw_ref[...]
        dis_ref[...] = dis
        disr_ref[...] = _rows8(dis, rows)
    return _tc1


def _make_tc2(rows):
    def _tc2(p_ref, z1_ref, dis_ref, mask_ref, w2_ref, b1_ref,
             z2_ref, z2r_ref):
        out1 = dis_ref[...] * (p_ref[0] + p_ref[1] + z1_ref[...]) + b1_ref[...]
        m = mask_ref[...]
        mp = jnp.concatenate(
            [m, jnp.zeros((out1.shape[0] - m.shape[0], m.shape[1]),
                          jnp.float32)])
        h1 = mp * jnp.maximum(out1, 0.0)
        z2 = dis_ref[...] * jnp.dot(h1, w2_ref[...],
                                    preferred_element_type=jnp.float32)
        z2_ref[...] = z2
        z2r_ref[...] = _rows8(z2, rows)
    return _tc2


def _tc3(qr_ref, z2r_ref, disr_ref, b2_ref, f1_ref, b1_ref, f2_ref, b2m_ref,
         o_ref):
    out2 = disr_ref[...] * (qr_ref[0] + qr_ref[1] + z2r_ref[...]) + b2_ref[...]
    t = jnp.maximum(out2, 0.0)
    r = jnp.maximum(jnp.dot(t, f1_ref[...],
                            preferred_element_type=jnp.float32) + b1_ref[...],
                    0.0)
    o_ref[...] = jnp.dot(r, f2_ref[...],
                         preferred_element_type=jnp.float32) + b2m_ref[...]


def kernel(x, edge_index, y, W1, b1, W2, b2, fc1_w, fc1_b, fc2_w, fc2_b):
    n, d = x.shape
    e = edge_index.shape[1]
    yn = y.shape[0]

    n_pad = ((n + NS * 16 - 1) // (NS * 16)) * (NS * 16)
    n_pad = ((n_pad + 127) // 128) * 128
    per_tile = ((e + NW * CHUNK - 1) // (NW * CHUNK)) * CHUNK
    e_pad = per_tile * NW

    pad_e = e_pad - e
    fill = jnp.full((2, pad_e), n, dtype=jnp.int32)
    edp = jnp.concatenate([edge_index, fill], axis=1).reshape(2, e_pad // 128,
                                                              128)
    ones_e = jnp.ones((CHUNK,), jnp.float32)
    zeros_n = jnp.zeros((n_pad,), jnp.float32)
    zeros_nf = jnp.zeros((n_pad, F), jnp.float32)
    keep = jax.random.bernoulli(jax.random.key(42), 0.6, (n, F))
    mask = jnp.where(keep, jnp.float32(1.0 / 0.6), jnp.float32(0.0))

    rows = tuple(1423 + 1431 * k for k in range(yn))
    deg_kernel = _make_deg_kernel(n_pad, e_pad)
    agg_kernel = _make_agg_kernel(n_pad, e_pad)
    agg2_kernel = _make_agg_kernel(n_pad, e_pad, out_rows=rows)

    deg16 = deg_kernel(edp, ones_e, zeros_n)

    xw = pl.pallas_call(
        functools.partial(_tc1a, n_pad),
        out_shape=jax.ShapeDtypeStruct((n_pad, F), jnp.float32),
    )(x, W1)

    z1, dis16, disr = pl.pallas_call(
        _make_tc1(rows),
        out_shape=[jax.ShapeDtypeStruct((n_pad, F), jnp.float32),
                   jax.ShapeDtypeStruct((n_pad, F), jnp.float32),
                   jax.ShapeDtypeStruct((8, F), jnp.float32)],
    )(xw, deg16)

    p = agg_kernel(edp, z1, zeros_nf)

    z2, z2r = pl.pallas_call(
        _make_tc2(rows),
        out_shape=[jax.ShapeDtypeStruct((n_pad, F), jnp.float32),
                   jax.ShapeDtypeStruct((8, F), jnp.float32)],
    )(p, z1, dis16, mask, W2, b1.reshape(1, F))

    qr = agg2_kernel(edp, z2, zeros_nf)

    f1b = jnp.broadcast_to(fc1_b.reshape(1, -1), (1, fc1_w.shape[1]))
    o = pl.pallas_call(
        _tc3,
        out_shape=jax.ShapeDtypeStruct((8, 1), jnp.float32),
    )(qr, z2r, disr, b2.reshape(1, F), fc1_w, f1b, fc2_w,
      fc2_b.reshape(1, 1))
    return o[:yn]
